# Initial kernel scaffold; baseline (speedup 1.0000x reference)
import jax
import jax.numpy as jnp
from jax.experimental import pallas as pl


def kernel(x, conv_weight):
    raise NotImplementedError("write your pallas kernel here")


# speedup vs baseline: 1.1381x; 1.1381x over previous
import functools

import jax
import jax.numpy as jnp
from jax.experimental import pallas as pl
from jax.experimental.pallas import tpu as pltpu

K = 7
P = 3


def _reduce_body(x_ref, avg_ref, max_ref, *, inv_c):
    v = x_ref[...]
    avg_ref[0] = jnp.sum(v, axis=1) * inv_c
    max_ref[0] = jnp.max(v, axis=1)


def _conv_body(avg_ref, max_ref, s_ref, o_ref, pad_ref, *, nb, h, w):
    hp, wp = h + 2 * P, w + 2 * P
    pad_ref[...] = jnp.zeros_like(pad_ref)
    pad_ref[:, P:P + h, P:P + w] = avg_ref[...]
    pad_ref[:, P:P + h, wp + P:wp + P + w] = max_ref[...]
    s = s_ref[...]
    for b in range(nb):
        t = jnp.dot(pad_ref[b], s, preferred_element_type=jnp.float32)
        acc = t[0:h, 0:w]
        for dy in range(1, K):
            acc = acc + t[dy:dy + h, dy * w:dy * w + w]
        o_ref[b] = jax.nn.sigmoid(acc)


def _band_matrix(conv_weight, w, wp):
    wm = conv_weight.reshape(2, K, K).astype(jnp.float32)
    ci = jnp.arange(wp)
    c = jnp.arange(w)
    dx = ci[:, None] - c[None, :]
    valid = (dx >= 0) & (dx < K)
    dxc = jnp.clip(dx, 0, K - 1)
    g = wm[:, :, dxc]
    g = jnp.where(valid[None, None], g, 0.0)
    return g.transpose(0, 2, 1, 3).reshape(2 * wp, K * w)


def kernel(x, conv_weight):
    N, C, H, W = x.shape
    HW = H * W
    Hp, Wp = H + 2 * P, W + 2 * P

    nb = 2 if N % 2 == 0 else 1
    nbc = 8 if N % 8 == 0 else 1

    x3 = x.reshape(N, C, HW)
    s_mat = _band_matrix(conv_weight, W, Wp)

    avg, mx = pl.pallas_call(
        functools.partial(_reduce_body, inv_c=1.0 / C),
        out_shape=(jax.ShapeDtypeStruct((N // nb, nb, HW), jnp.float32),
                   jax.ShapeDtypeStruct((N // nb, nb, HW), jnp.float32)),
        grid=(N // nb,),
        in_specs=[pl.BlockSpec((nb, C, HW), lambda i: (i, 0, 0))],
        out_specs=(pl.BlockSpec((1, nb, HW), lambda i: (i, 0, 0)),
                   pl.BlockSpec((1, nb, HW), lambda i: (i, 0, 0))),
        compiler_params=pltpu.CompilerParams(
            dimension_semantics=("parallel",),
            vmem_limit_bytes=48 << 20),
        cost_estimate=pl.CostEstimate(
            flops=2 * N * C * HW,
            transcendentals=0,
            bytes_accessed=(N * C * HW + 2 * N * HW) * 4),
    )(x3)

    out = pl.pallas_call(
        functools.partial(_conv_body, nb=nbc, h=H, w=W),
        out_shape=jax.ShapeDtypeStruct((N, H, W), x.dtype),
        grid=(N // nbc,),
        in_specs=[
            pl.BlockSpec((nbc, H, W), lambda i: (i, 0, 0)),
            pl.BlockSpec((nbc, H, W), lambda i: (i, 0, 0)),
            pl.BlockSpec((2 * Wp, K * W), lambda i: (0, 0)),
        ],
        out_specs=pl.BlockSpec((nbc, H, W), lambda i: (i, 0, 0)),
        scratch_shapes=[pltpu.VMEM((nbc, Hp, 2 * Wp), jnp.float32)],
        compiler_params=pltpu.CompilerParams(
            dimension_semantics=("parallel",),
            vmem_limit_bytes=32 << 20),
        cost_estimate=pl.CostEstimate(
            flops=2 * N * Hp * 2 * Wp * K * W + 8 * N * HW,
            transcendentals=N * HW,
            bytes_accessed=(3 * N * HW + 2 * Wp * K * W) * 4),
    )(avg.reshape(N, H, W), mx.reshape(N, H, W), s_mat)

    return out.reshape(N, 1, H, W)
